# baseline (speedup 1.0000x reference)
import numpy as np
import jax
import jax.numpy as jnp
from jax.experimental import pallas as pl
from jax.experimental.pallas import tpu as pltpu

D = 256
M = 64
T = 20
NUM_CLASSES = 10


def _band_coeffs():
    offsets = np.arange(-2, 3)
    wslot = np.zeros((T, M), dtype=np.float64)
    for t in range(T):
        idx = (t + offsets) % M
        delta = idx.astype(np.float32) - np.float32(t)
        logits = (-(delta.astype(np.float32) ** 2) / np.float32(8.0)).astype(np.float32)
        e = np.exp(logits).astype(np.float32)
        w = (e / e.sum(dtype=np.float32)).astype(np.float32)
        wslot[t, idx] = w
    return np.tril(wslot @ wslot.T, -1)


_C = _band_coeffs()


def _recurrence_kernel(x_ref, p_ref, out_ref):
    x = x_ref[...]
    wu = p_ref[0:D, :]
    we = p_ref[D + 10:D + 11, :]
    be = p_ref[D + 11:D + 12, :]
    bu = p_ref[D + 12:D + 13, :]
    gamma = p_ref[D + 13:D + 14, :]
    beta = p_ref[D + 14:D + 15, :]
    cs = jax.nn.sigmoid(p_ref[D + 16, 0])

    B = x.shape[0]
    h = jnp.zeros((B, D), jnp.float32)
    hist = []
    for t in range(T):
        inp = jnp.tanh(x[:, t:t + 1] * we + be)
        pre_in = inp
        for d in range(1, 5):
            u = t - d
            if u < 0:
                continue
            coeff = cs * np.float32(_C[t, u]) if _C[t, u] != 0.0 else None
            if d == 1:
                coeff = coeff + np.float32(1.0) if coeff is not None else None
                pre_in = pre_in + (hist[u] if coeff is None else coeff * hist[u])
            elif coeff is not None:
                pre_in = pre_in + coeff * hist[u]
        pre = jax.lax.dot_general(
            pre_in, wu, (((1,), (1,)), ((), ())),
            preferred_element_type=jnp.float32) + bu
        ht = jnp.tanh(pre)
        mu = jnp.mean(ht, axis=1, keepdims=True)
        var = jnp.mean((ht - mu) * (ht - mu), axis=1, keepdims=True)
        ht = (ht - mu) * jax.lax.rsqrt(var + 1e-5) * gamma + beta
        h = ht
        hist.append(ht)

    wo = p_ref[D:D + NUM_CLASSES, :]
    bo = p_ref[D + 15:D + 16, 0:NUM_CLASSES]
    out_ref[...] = jax.lax.dot_general(
        h, wo, (((1,), (1,)), ((), ())),
        preferred_element_type=jnp.float32) + bo


def kernel(x, W_embed, b_embed, W_update, b_update, gamma, beta, W_out,
           b_out, ctx_strength):
    B = x.shape[0]
    x2 = x.reshape(B, T)
    packed = jnp.concatenate([
        W_update,
        W_out,
        W_embed.reshape(1, D),
        b_embed.reshape(1, D),
        b_update.reshape(1, D),
        gamma.reshape(1, D),
        beta.reshape(1, D),
        jnp.pad(b_out, (0, D - NUM_CLASSES)).reshape(1, D),
        jnp.broadcast_to(ctx_strength, (1, D)),
    ], axis=0)

    return pl.pallas_call(
        _recurrence_kernel,
        out_shape=jax.ShapeDtypeStruct((B, NUM_CLASSES), jnp.float32),
    )(x2, packed)


# speedup vs baseline: 1.2181x; 1.2181x over previous
import numpy as np
import jax
import jax.numpy as jnp
from jax.experimental import pallas as pl
from jax.experimental.pallas import tpu as pltpu

D = 256
M = 64
T = 20
NUM_CLASSES = 10


def _band_coeffs():
    offsets = np.arange(-2, 3)
    wslot = np.zeros((T, M), dtype=np.float64)
    for t in range(T):
        idx = (t + offsets) % M
        delta = idx.astype(np.float32) - np.float32(t)
        logits = (-(delta.astype(np.float32) ** 2) / np.float32(8.0)).astype(np.float32)
        e = np.exp(logits).astype(np.float32)
        w = (e / e.sum(dtype=np.float32)).astype(np.float32)
        wslot[t, idx] = w
    return np.tril(wslot @ wslot.T, -1)


_C = _band_coeffs()


def _recurrence_kernel(x_ref, we_ref, be_ref, wu_ref, bu_ref, gamma_ref,
                       beta_ref, wo_ref, bo_ref, cs_ref, out_ref):
    x = x_ref[...]
    we = we_ref[...]
    be = be_ref[...]
    wu = wu_ref[...]
    bu = bu_ref[...]
    gamma = gamma_ref[...]
    beta = beta_ref[...]
    cs = jax.nn.sigmoid(cs_ref[0, 0])

    B = x.shape[0]

    def a1(t):
        return np.float32(1.0) + cs * np.float32(_C[t, t - 1]) if t >= 1 else None

    scale = [a1(u + 1) if u + 1 < T else a1(T - 1) for u in range(T)]
    gs = {}
    for u in range(T):
        key = float(_C[u + 1, u]) if u + 1 < T else float(_C[T - 1, T - 2])
        if key not in gs:
            gs[key] = (scale[u] * gamma, scale[u] * beta)

    def affine(u):
        key = float(_C[u + 1, u]) if u + 1 < T else float(_C[T - 1, T - 2])
        return gs[key]

    hist = []
    for t in range(T):
        inp = jnp.tanh(x[:, t:t + 1] * we + be)
        pre_in = inp
        for d in range(1, 5):
            u = t - d
            if u < 0 or _C[t, u] == 0.0:
                continue
            if d == 1:
                pre_in = pre_in + hist[u]
            else:
                ratio = (cs * np.float32(_C[t, u])) / scale[u]
                pre_in = pre_in + ratio * hist[u]
        pre = jax.lax.dot_general(
            pre_in, wu, (((1,), (1,)), ((), ())),
            preferred_element_type=jnp.float32) + bu
        ht = jnp.tanh(pre)
        mu = jnp.mean(ht, axis=1, keepdims=True)
        var = jnp.mean((ht - mu) * (ht - mu), axis=1, keepdims=True)
        g_u, b_u = affine(t)
        hist.append((ht - mu) * jax.lax.rsqrt(var + 1e-5) * g_u + b_u)

    wo = wo_ref[...] * (np.float32(1.0) / scale[T - 1])
    bo = bo_ref[...]
    out_ref[...] = jax.lax.dot_general(
        hist[T - 1], wo, (((1,), (1,)), ((), ())),
        preferred_element_type=jnp.float32) + bo


def kernel(x, W_embed, b_embed, W_update, b_update, gamma, beta, W_out,
           b_out, ctx_strength):
    B = x.shape[0]
    x2 = x.reshape(B, T)
    we = W_embed.reshape(1, D)
    be = b_embed.reshape(1, D)
    bu = b_update.reshape(1, D)
    g = gamma.reshape(1, D)
    bt = beta.reshape(1, D)
    bo = b_out.reshape(1, NUM_CLASSES)
    cs = jnp.reshape(ctx_strength, (1, 1))

    return pl.pallas_call(
        _recurrence_kernel,
        out_shape=jax.ShapeDtypeStruct((B, NUM_CLASSES), jnp.float32),
    )(x2, we, be, W_update, bu, g, bt, W_out, bo, cs)


# speedup vs baseline: 1.2267x; 1.0070x over previous
import numpy as np
import jax
import jax.numpy as jnp
from jax.experimental import pallas as pl
from jax.experimental.pallas import tpu as pltpu

D = 256
M = 64
T = 20
NUM_CLASSES = 10


def _band_coeffs():
    offsets = np.arange(-2, 3)
    wslot = np.zeros((T, M), dtype=np.float64)
    for t in range(T):
        idx = (t + offsets) % M
        delta = idx.astype(np.float32) - np.float32(t)
        logits = (-(delta.astype(np.float32) ** 2) / np.float32(8.0)).astype(np.float32)
        e = np.exp(logits).astype(np.float32)
        w = (e / e.sum(dtype=np.float32)).astype(np.float32)
        wslot[t, idx] = w
    return np.tril(wslot @ wslot.T, -1)


_C = _band_coeffs()


STEADY_T = 6


def _recurrence_kernel(x_ref, we_ref, be_ref, wu_ref, bu_ref, gamma_ref,
                       beta_ref, wo_ref, bo_ref, cs_ref, out_ref,
                       hbuf_ref, wtail_ref):
    x = x_ref[...]
    we = we_ref[...]
    be = be_ref[...]
    wu = wu_ref[...]
    bu = bu_ref[...]
    gamma = gamma_ref[...]
    beta = beta_ref[...]
    cs = jax.nn.sigmoid(cs_ref[0, 0])

    B = x.shape[0]

    def a1(t):
        return np.float32(1.0) + cs * np.float32(_C[t, t - 1]) if t >= 1 else None

    scale = [a1(u + 1) if u + 1 < T else a1(T - 1) for u in range(T)]
    gs = {}
    for u in range(T):
        key = float(_C[u + 1, u]) if u + 1 < T else float(_C[T - 1, T - 2])
        if key not in gs:
            gs[key] = (scale[u] * gamma, scale[u] * beta)

    def affine(u):
        key = float(_C[u + 1, u]) if u + 1 < T else float(_C[T - 1, T - 2])
        return gs[key]

    wut = wu.T
    for k, d in enumerate((4, 3, 2)):
        ratio = (cs * np.float32(_C[STEADY_T, STEADY_T - d])) / scale[2]
        wtail_ref[k * D:(k + 1) * D, :] = ratio * wut

    hist = []
    for t in range(T):
        inp = jnp.tanh(x[:, t:t + 1] * we + be)
        if t < STEADY_T:
            pre_in = inp
            for d in range(1, 5):
                u = t - d
                if u < 0 or _C[t, u] == 0.0:
                    continue
                if d == 1:
                    pre_in = pre_in + hist[u]
                else:
                    ratio = (cs * np.float32(_C[t, u])) / scale[u]
                    pre_in = pre_in + ratio * hist[u]
            pre = jax.lax.dot_general(
                pre_in, wu, (((1,), (1,)), ((), ())),
                preferred_element_type=jnp.float32) + bu
        else:
            tail = jax.lax.dot_general(
                hbuf_ref[:, (t - 4) * D:(t - 1) * D], wtail_ref[...],
                (((1,), (0,)), ((), ())),
                preferred_element_type=jnp.float32)
            pre = jax.lax.dot_general(
                inp + hist[t - 1], wu, (((1,), (1,)), ((), ())),
                preferred_element_type=jnp.float32) + (tail + bu)
        ht = jnp.tanh(pre)
        mu = jnp.mean(ht, axis=1, keepdims=True)
        var = jnp.mean((ht - mu) * (ht - mu), axis=1, keepdims=True)
        g_u, b_u = affine(t)
        hs = (ht - mu) * jax.lax.rsqrt(var + 1e-5) * g_u + b_u
        hist.append(hs)
        if 2 <= t <= T - 3:
            hbuf_ref[:, t * D:(t + 1) * D] = hs

    wo = wo_ref[...] * (np.float32(1.0) / scale[T - 1])
    bo = bo_ref[...]
    out_ref[...] = jax.lax.dot_general(
        hist[T - 1], wo, (((1,), (1,)), ((), ())),
        preferred_element_type=jnp.float32) + bo


def kernel(x, W_embed, b_embed, W_update, b_update, gamma, beta, W_out,
           b_out, ctx_strength):
    B = x.shape[0]
    x2 = x.reshape(B, T)
    we = W_embed.reshape(1, D)
    be = b_embed.reshape(1, D)
    bu = b_update.reshape(1, D)
    g = gamma.reshape(1, D)
    bt = beta.reshape(1, D)
    bo = b_out.reshape(1, NUM_CLASSES)
    cs = jnp.reshape(ctx_strength, (1, 1))

    return pl.pallas_call(
        _recurrence_kernel,
        out_shape=jax.ShapeDtypeStruct((B, NUM_CLASSES), jnp.float32),
        scratch_shapes=[
            pltpu.VMEM((B, T * D), jnp.float32),
            pltpu.VMEM((3 * D, D), jnp.float32),
        ],
    )(x2, we, be, W_update, bu, g, bt, W_out, bo, cs)
